# baseline (speedup 1.0000x reference)
import functools

import jax
import jax.numpy as jnp
from jax import lax
from jax.experimental import pallas as pl
from jax.experimental.pallas import tpu as pltpu
from jax.experimental.pallas import tpu_sc as plsc

_NW = 32
_CHUNK = 128
_D = 32


@functools.partial(jax.jit, static_argnums=(4,))
def _run(idx1, idx2, w1, w2, n_chunks_per_w):
    total_chunks = _NW * n_chunks_per_w
    mesh = plsc.VectorSubcoreMesh(core_axis_name="c", subcore_axis_name="s")

    @functools.partial(
        pl.kernel,
        mesh=mesh,
        out_type=jax.ShapeDtypeStruct((total_chunks, _CHUNK, _D), jnp.float32),
        compiler_params=pltpu.CompilerParams(use_tc_tiling_on_sc=False),
        scratch_types=[
            pltpu.VMEM((n_chunks_per_w, _CHUNK), jnp.int32),
            pltpu.VMEM((n_chunks_per_w, _CHUNK), jnp.int32),
            pltpu.VMEM((_CHUNK, _D), jnp.float32),
            pltpu.VMEM((_CHUNK, _D), jnp.float32),
            pltpu.SemaphoreType.DMA,
            pltpu.SemaphoreType.DMA,
        ],
    )
    def k(idx1_hbm, idx2_hbm, w1_hbm, w2_hbm, out_hbm,
          i1_v, i2_v, buf_a, buf_b, sem_a, sem_b):
        cid = lax.axis_index("c")
        sid = lax.axis_index("s")
        wid = sid * 2 + cid
        pltpu.sync_copy(idx1_hbm.at[wid], i1_v)
        pltpu.sync_copy(idx2_hbm.at[wid], i2_v)

        def body(c, carry):
            gc = wid * n_chunks_per_w + c
            cp_a = pltpu.async_copy(w1_hbm.at[i1_v.at[c]], buf_a, sem_a)
            cp_a.wait()
            cp_b = pltpu.async_copy(w2_hbm.at[i2_v.at[c]], buf_a, sem_b, add=True)
            cp_b.wait()
            pltpu.sync_copy(buf_a, out_hbm.at[gc])
            return carry

        lax.fori_loop(0, n_chunks_per_w, body, 0)

    return k(idx1, idx2, w1, w2)


def kernel(input, another_input, W1, W2):
    B, H = input.shape
    total = B * H
    n_chunks_per_w = total // (_NW * _CHUNK)
    idx1 = input.reshape(_NW, n_chunks_per_w, _CHUNK).astype(jnp.int32)
    idx2 = another_input.reshape(_NW, n_chunks_per_w, _CHUNK).astype(jnp.int32)
    out = _run(idx1, idx2, W1, W2, n_chunks_per_w)
    return out.reshape(B, H, _D)


# speedup vs baseline: 1.1483x; 1.1483x over previous
import functools

import jax
import jax.numpy as jnp
from jax import lax
from jax.experimental import pallas as pl
from jax.experimental.pallas import tpu as pltpu
from jax.experimental.pallas import tpu_sc as plsc

_NW = 32
_CHUNK = 128
_D = 32
_G = 8


@functools.partial(jax.jit, static_argnums=(4,))
def _run(idx1, idx2, w1, w2, n_chunks_per_w):
    total_chunks = _NW * n_chunks_per_w
    n_groups = n_chunks_per_w // _G
    mesh = plsc.VectorSubcoreMesh(core_axis_name="c", subcore_axis_name="s")

    @functools.partial(
        pl.kernel,
        mesh=mesh,
        out_type=jax.ShapeDtypeStruct((total_chunks, _CHUNK, _D), jnp.float32),
        compiler_params=pltpu.CompilerParams(use_tc_tiling_on_sc=False),
        scratch_types=[
            pltpu.VMEM((n_chunks_per_w, _CHUNK), jnp.int32),
            pltpu.VMEM((n_chunks_per_w, _CHUNK), jnp.int32),
            pltpu.VMEM((_G, _CHUNK, _D), jnp.float32),
            pltpu.SemaphoreType.DMA((_G,)),
        ],
    )
    def k(idx1_hbm, idx2_hbm, w1_hbm, w2_hbm, out_hbm, i1_v, i2_v, buf, sems):
        cid = lax.axis_index("c")
        sid = lax.axis_index("s")
        wid = sid * 2 + cid
        base = wid * n_chunks_per_w
        pltpu.sync_copy(idx1_hbm.at[wid], i1_v)
        pltpu.sync_copy(idx2_hbm.at[wid], i2_v)

        for j in range(_G):
            pltpu.async_copy(w1_hbm.at[i1_v.at[j]], buf.at[j], sems.at[j])

        def group_body(g, carry):
            for j in range(_G):
                c = g * _G + j
                pltpu.make_async_copy(
                    w1_hbm.at[i1_v.at[c]], buf.at[j], sems.at[j]).wait()
                pltpu.async_copy(
                    w2_hbm.at[i2_v.at[c]], buf.at[j], sems.at[j], add=True)
            for j in range(_G):
                c = g * _G + j
                pltpu.make_async_copy(
                    w2_hbm.at[i2_v.at[c]], buf.at[j], sems.at[j]).wait()
                pltpu.async_copy(buf.at[j], out_hbm.at[base + c], sems.at[j])
            for j in range(_G):
                c = g * _G + j
                pltpu.make_async_copy(
                    buf.at[j], out_hbm.at[base + c], sems.at[j]).wait()

                @pl.when(g + 1 < n_groups)
                def _():
                    c2 = (g + 1) * _G + j
                    pltpu.async_copy(
                        w1_hbm.at[i1_v.at[c2]], buf.at[j], sems.at[j])
            return carry

        lax.fori_loop(0, n_groups, group_body, 0)

    return k(idx1, idx2, w1, w2)


def kernel(input, another_input, W1, W2):
    B, H = input.shape
    total = B * H
    n_chunks_per_w = total // (_NW * _CHUNK)
    idx1 = input.reshape(_NW, n_chunks_per_w, _CHUNK).astype(jnp.int32)
    idx2 = another_input.reshape(_NW, n_chunks_per_w, _CHUNK).astype(jnp.int32)
    out = _run(idx1, idx2, W1, W2, n_chunks_per_w)
    return out.reshape(B, H, _D)
